# baseline (speedup 1.0000x reference)
import functools

import jax
import jax.numpy as jnp
from jax import lax
from jax.experimental import pallas as pl
from jax.experimental.pallas import tpu as pltpu
from jax.experimental.pallas import tpu_sc as plsc


def _sc_gather_elems(pe_flat, eidx, n_el, nw, nc):
    per_w = n_el // nw
    mesh = plsc.VectorSubcoreMesh(core_axis_name="c", subcore_axis_name="s")

    @functools.partial(
        pl.kernel,
        mesh=mesh,
        out_type=jax.ShapeDtypeStruct((n_el,), jnp.float32),
        compiler_params=pltpu.CompilerParams(use_tc_tiling_on_sc=False),
        scratch_types=[
            pltpu.VMEM((per_w,), jnp.int32),
            pltpu.VMEM((per_w,), jnp.float32),
            pltpu.SemaphoreType.DMA,
        ],
    )
    def gather_kernel(eidx_hbm, table_hbm, out_hbm, eidx_v, vals_v, sem):
        wid = lax.axis_index("s") * nc + lax.axis_index("c")
        base = wid * per_w
        pltpu.sync_copy(eidx_hbm.at[pl.ds(base, per_w)], eidx_v)
        pltpu.async_copy(table_hbm.at[eidx_v], vals_v, sem).wait()
        pltpu.sync_copy(vals_v, out_hbm.at[pl.ds(base, per_w)])

    return gather_kernel(eidx, pe_flat)


def _add_body(p_ref, x_ref, o_ref):
    o_ref[...] = x_ref[...] + p_ref[...][:, :, None]


def kernel(input, indices, pe):
    b, l, d = input.shape
    v = pe.shape[0]
    info = plsc.get_sparse_core_info()
    nc, ns = info.num_cores, info.num_subcores
    nw = nc * ns

    align = 8 * nw
    l_pad = ((l + align - 1) // align) * align
    idx_padded = jnp.pad(indices.astype(jnp.int32), (0, l_pad - l))

    ct = v // 128
    pe_flat = (
        jnp.transpose(pe)
        .reshape(d // 8, 8, ct, 128)
        .transpose(0, 2, 1, 3)
        .reshape(-1)
    )
    base = (idx_padded >> 7) * 1024 + (idx_padded & 127)
    dd = jnp.arange(d, dtype=jnp.int32)
    off_d = (dd >> 3) * (ct * 1024) + (dd & 7) * 128
    eidx = (base[:, None] + off_d[None, :]).reshape(-1)

    rows = _sc_gather_elems(pe_flat, eidx, l_pad * d, nw, nc).reshape(l_pad, d)

    x_t = jnp.transpose(input, (1, 2, 0))
    lt, bt = 8, 2048
    out_t = pl.pallas_call(
        _add_body,
        grid=(l // lt, b // bt),
        in_specs=[
            pl.BlockSpec((lt, d), lambda i, j: (i, 0)),
            pl.BlockSpec((lt, d, bt), lambda i, j: (i, 0, j)),
        ],
        out_specs=pl.BlockSpec((lt, d, bt), lambda i, j: (i, 0, j)),
        out_shape=jax.ShapeDtypeStruct((l, d, b), jnp.float32),
    )(rows[:l], x_t)
    return jnp.transpose(out_t, (2, 0, 1))


# speedup vs baseline: 1.0206x; 1.0206x over previous
import functools

import jax
import jax.numpy as jnp
from jax import lax
from jax.experimental import pallas as pl
from jax.experimental.pallas import tpu as pltpu
from jax.experimental.pallas import tpu_sc as plsc


def _sc_gather_rows(pe_flat, eidx, n_rows, nc):
    n_el = n_rows * 128
    active = n_rows // 8
    per_w = 8 * 128
    mesh = plsc.VectorSubcoreMesh(core_axis_name="c", subcore_axis_name="s")

    @functools.partial(
        pl.kernel,
        mesh=mesh,
        out_type=jax.ShapeDtypeStruct((n_el,), jnp.float32),
        compiler_params=pltpu.CompilerParams(use_tc_tiling_on_sc=False),
        scratch_types=[
            pltpu.VMEM((per_w,), jnp.int32),
            pltpu.VMEM((per_w,), jnp.float32),
            pltpu.SemaphoreType.DMA,
        ],
    )
    def gather_kernel(eidx_hbm, table_hbm, out_hbm, eidx_v, vals_v, sem):
        wid = lax.axis_index("s") * nc + lax.axis_index("c")

        @pl.when(wid < active)
        def _():
            base = wid * per_w
            pltpu.sync_copy(eidx_hbm.at[pl.ds(base, per_w)], eidx_v)
            pltpu.async_copy(table_hbm.at[eidx_v], vals_v, sem).wait()
            pltpu.sync_copy(vals_v, out_hbm.at[pl.ds(base, per_w)])

    return gather_kernel(eidx, pe_flat)


def _add_body(p_ref, x_ref, o_ref):
    d = x_ref.shape[1]
    o_ref[...] = x_ref[...] + p_ref[...][:, :d, None]


def kernel(input, indices, pe):
    b, l, d = input.shape
    v = pe.shape[0]
    info = plsc.get_sparse_core_info()
    nc = info.num_cores

    ct = v // 128
    pe_flat = (
        jnp.transpose(pe)
        .reshape(d // 8, 8, ct, 128)
        .transpose(0, 2, 1, 3)
        .reshape(-1)
    )
    idx = indices.astype(jnp.int32)
    base = (idx >> 7) * 1024 + (idx & 127)
    ddp = jnp.arange(128, dtype=jnp.int32)
    off_d = jnp.where(ddp < d, (ddp >> 3) * (ct * 1024) + (ddp & 7) * 128, 0)
    eidx = (base[:, None] + off_d[None, :]).reshape(-1)

    p_pad = _sc_gather_rows(pe_flat, eidx, l, nc).reshape(l, 128)

    x_t = jnp.transpose(input, (1, 2, 0))
    lt, bt = 8, 4096
    out_t = pl.pallas_call(
        _add_body,
        grid=(l // lt, b // bt),
        in_specs=[
            pl.BlockSpec((lt, 128), lambda i, j: (i, 0)),
            pl.BlockSpec((lt, d, bt), lambda i, j: (i, 0, j)),
        ],
        out_specs=pl.BlockSpec((lt, d, bt), lambda i, j: (i, 0, j)),
        out_shape=jax.ShapeDtypeStruct((l, d, b), jnp.float32),
    )(p_pad, x_t)
    return jnp.transpose(out_t, (2, 0, 1))
